# Initial kernel scaffold; baseline (speedup 1.0000x reference)
import jax
import jax.numpy as jnp
from jax.experimental import pallas as pl


def kernel(edge_index, num_nodes):
    raise NotImplementedError("write your pallas kernel here")


# speedup vs baseline: 1.3933x; 1.3933x over previous
import functools

import jax
import jax.numpy as jnp
from jax import lax
from jax.experimental import pallas as pl
from jax.experimental.pallas import tpu as pltpu
from jax.experimental.pallas import tpu_sc as plsc

NUM_STEPS = 8
KEEP_COLS = (2, 4, 8)
N_STATIC = 50000
E = 1600000

NC = 2
NS = 16
NW = NC * NS

NB = 50176
NSL = NB // NW
TE = NB
E_PAD = NW * TE
EW = 3584
NWIN = TE // EW
ECH = 8
WT = NSL
WCH = 112
WCN = WT // WCH

_mesh = plsc.VectorSubcoreMesh(core_axis_name="c", subcore_axis_name="s")
_cparams = pltpu.CompilerParams(needs_layout_passes=False)
_ones16 = lambda: jnp.ones((16,), jnp.int32)


def _wid():
  return lax.axis_index("s") * NC + lax.axis_index("c")


def _hist_body(rows_hbm, hist_hbm, tots_hbm, hv, wv, tv, sem):
  w = _wid()
  base_e = w * TE

  def zero(i, _):
    hv[pl.ds(i * 16, 16)] = jnp.zeros((16,), jnp.int32)
    return ()
  lax.fori_loop(0, NB // 16, zero, ())

  def win(wi, _):
    pltpu.sync_copy(rows_hbm.at[pl.ds(base_e + wi * EW, EW)], wv)

    def vstep(j, _):
      r = wv[pl.ds(j * 16, 16)]
      plsc.addupdate_scatter(hv, [r], _ones16())
      return ()
    lax.fori_loop(0, EW // 16, vstep, ())
    return ()
  lax.fori_loop(0, NWIN, win, ())

  lane = lax.iota(jnp.int32, 16)

  def tot(t, _):
    def acc(j, a):
      return a + hv[pl.ds(t * NSL + j * 16, 16)]
    a = lax.fori_loop(0, NSL // 16, acc, jnp.zeros((16,), jnp.int32))
    total = jnp.sum(a)
    plsc.store_scatter(tv, [jnp.full((16,), t, jnp.int32)],
                       jnp.full((16,), total, jnp.int32),
                       mask=lane == lax.rem(t, 16))
    return ()
  lax.fori_loop(0, NW, tot, ())

  pltpu.sync_copy(hv, hist_hbm.at[pl.ds(w * NB, NB)])
  pltpu.sync_copy(tv, tots_hbm.at[pl.ds(w * NW, NW)])


_hist = pl.kernel(
    _hist_body, mesh=_mesh, compiler_params=_cparams,
    out_type=(jax.ShapeDtypeStruct((NW * NB,), jnp.int32),
              jax.ShapeDtypeStruct((NW * NW,), jnp.int32)),
    scratch_types=[pltpu.VMEM((NB,), jnp.int32),
                   pltpu.VMEM((EW,), jnp.int32),
                   pltpu.VMEM((NW,), jnp.int32),
                   pltpu.SemaphoreType.DMA],
)


def _base_body(hist_hbm, tots_hbm, rowptr_hbm, b_hbm, hbuf, tbuf, degbuf, rpbuf,
               sem):
  t = _wid()
  pltpu.sync_copy(tots_hbm, tbuf)
  lane = lax.iota(jnp.int32, 16)

  def acc_t(w, c):
    return (c[0] + tbuf[pl.ds(w * NW, 16)], c[1] + tbuf[pl.ds(w * NW + 16, 16)])
  nt0, nt1 = lax.fori_loop(0, NW, acc_t,
                           (jnp.zeros((16,), jnp.int32),
                            jnp.zeros((16,), jnp.int32)))
  zero16 = jnp.zeros((16,), jnp.int32)
  grand = (jnp.sum(jnp.where(lane < t, nt0, zero16)) +
           jnp.sum(jnp.where(lane + 16 < t, nt1, zero16)))

  def load_h(w, _):
    pltpu.sync_copy(hist_hbm.at[pl.ds(w * NB + t * NSL, NSL)],
                    hbuf.at[pl.ds(w * NSL, NSL)])
    return ()
  lax.fori_loop(0, NW, load_h, ())

  def zero_deg(j, _):
    degbuf[pl.ds(j * 16, 16)] = zero16
    return ()
  lax.fori_loop(0, NSL // 16, zero_deg, ())

  def acc_deg(w, _):
    def inner(j, _):
      degbuf[pl.ds(j * 16, 16)] += hbuf[pl.ds(w * NSL + j * 16, 16)]
      return ()
    lax.fori_loop(0, NSL // 16, inner, ())
    return ()
  lax.fori_loop(0, NW, acc_deg, ())

  def scan(j, carry):
    v = degbuf[pl.ds(j * 16, 16)]
    c = plsc.cumsum(v)
    rpbuf[pl.ds(j * 16, 16)] = c - v + carry
    return carry + jnp.sum(v)
  lax.fori_loop(0, NSL // 16, scan, grand)
  pltpu.sync_copy(rpbuf, rowptr_hbm.at[pl.ds(t * NSL, NSL)])

  def emit(w, _):
    pltpu.sync_copy(rpbuf, b_hbm.at[pl.ds(w * NB + t * NSL, NSL)])

    def inner(j, _):
      rpbuf[pl.ds(j * 16, 16)] += hbuf[pl.ds(w * NSL + j * 16, 16)]
      return ()
    lax.fori_loop(0, NSL // 16, inner, ())
    return ()
  lax.fori_loop(0, NW, emit, ())


_base = pl.kernel(
    _base_body, mesh=_mesh, compiler_params=_cparams,
    out_type=(jax.ShapeDtypeStruct((NB,), jnp.int32),
              jax.ShapeDtypeStruct((NW * NB,), jnp.int32)),
    scratch_types=[pltpu.VMEM((NW * NSL,), jnp.int32),
                   pltpu.VMEM((NW * NW,), jnp.int32),
                   pltpu.VMEM((NSL,), jnp.int32),
                   pltpu.VMEM((NSL,), jnp.int32),
                   pltpu.SemaphoreType.DMA],
)


def _perm_body(rows_hbm, cols_hbm, b_hbm, cs_hbm, bbuf, rwin, cwin, pbuf, svbuf,
               sem):
  w = _wid()
  base_e = w * TE
  pltpu.sync_copy(b_hbm.at[pl.ds(w * NB, NB)], bbuf)
  lane = lax.iota(jnp.int32, 16)
  lane_m1 = jnp.maximum(lane - 1, 0)

  def win(wi, _):
    off = base_e + wi * EW
    pltpu.sync_copy(rows_hbm.at[pl.ds(off, EW)], rwin)
    pltpu.sync_copy(cols_hbm.at[pl.ds(off, EW)], cwin)

    def chunk(c, _):
      for jj in range(ECH):
        r = rwin[pl.ds(c * 128 + jj * 16, 16)]
        cv = cwin[pl.ds(c * 128 + jj * 16, 16)]
        key = r * 16 + lane
        sk, scol = plsc.sort_key_val(key, cv)
        rs = lax.shift_right_logical(sk, 4)
        prev = jnp.take(rs, lane_m1)
        neq = jnp.logical_or(lane == 0, rs != prev)
        start = plsc.cummax(jnp.where(neq, lane, jnp.zeros((16,), jnp.int32)))
        rank = lane - start
        b = plsc.load_gather(bbuf, [rs])
        plsc.addupdate_scatter(bbuf, [rs], _ones16())
        pbuf[c, pl.ds(jj * 16, 16)] = b + rank
        svbuf[c, pl.ds(jj * 16, 16)] = scol
      pltpu.async_copy(svbuf.at[c], cs_hbm.at[pbuf.at[c]], sem)
      return ()
    lax.fori_loop(0, EW // 128, chunk, ())

    def drain(c, _):
      pltpu.make_async_copy(svbuf.at[c], cs_hbm.at[pbuf.at[c]], sem).wait()
      return ()
    lax.fori_loop(0, EW // 128, drain, ())
    return ()
  lax.fori_loop(0, NWIN, win, ())


_perm = pl.kernel(
    _perm_body, mesh=_mesh, compiler_params=_cparams,
    out_type=jax.ShapeDtypeStruct((E_PAD,), jnp.int32),
    scratch_types=[pltpu.VMEM((NB,), jnp.int32),
                   pltpu.VMEM((EW,), jnp.int32),
                   pltpu.VMEM((EW,), jnp.int32),
                   pltpu.VMEM((EW // 128, 128), jnp.int32),
                   pltpu.VMEM((EW // 128, 128), jnp.int32),
                   pltpu.SemaphoreType.DMA],
)


def _walk_body(rowptr_hbm, cs_hbm, u_hbm, starts_hbm, out_hbm, rpv, uv, curv,
               dv, idxv, gv, sem):
  w = _wid()
  base_w = w * WT
  pltpu.sync_copy(rowptr_hbm, rpv)
  pltpu.sync_copy(starts_hbm.at[pl.ds(base_w, WT)], curv)
  out_row = 0
  for s in range(NUM_STEPS):
    pltpu.sync_copy(u_hbm.at[pl.ds(s * NB + base_w, WT)], uv)

    def chunk(c, _):
      for jj in range(WCH // 16):
        o = c * WCH + jj * 16
        cur = curv[pl.ds(o, 16)]
        r0 = plsc.load_gather(rpv, [cur])
        r1 = plsc.load_gather(rpv, [cur + 1])
        d = r1 - r0
        u = uv[pl.ds(o, 16)]
        off = (u * d.astype(jnp.float32)).astype(jnp.int32)
        off = jnp.minimum(off, jnp.maximum(d - 1, jnp.zeros((16,), jnp.int32)))
        dv[pl.ds(o, 16)] = d
        idxv[c, pl.ds(jj * 16, 16)] = jnp.where(d > 0, r0 + off,
                                                jnp.zeros((16,), jnp.int32))
      pltpu.async_copy(cs_hbm.at[idxv.at[c]], gv.at[c], sem)
      return ()
    lax.fori_loop(0, WCN, chunk, ())

    def drain(c, _):
      pltpu.make_async_copy(cs_hbm.at[idxv.at[c]], gv.at[c], sem).wait()
      return ()
    lax.fori_loop(0, WCN, drain, ())

    def sel(c, _):
      for jj in range(WCH // 16):
        o = c * WCH + jj * 16
        d = dv[pl.ds(o, 16)]
        g = gv[c, pl.ds(jj * 16, 16)]
        cur = curv[pl.ds(o, 16)]
        curv[pl.ds(o, 16)] = jnp.where(d > 0, g, cur)
      return ()
    lax.fori_loop(0, WCN, sel, ())

    if (s + 1) in KEEP_COLS:
      pltpu.sync_copy(curv, out_hbm.at[pl.ds(out_row * NB + base_w, WT)])
      out_row += 1


_walk = pl.kernel(
    _walk_body, mesh=_mesh, compiler_params=_cparams,
    out_type=jax.ShapeDtypeStruct((len(KEEP_COLS) * NB,), jnp.int32),
    scratch_types=[pltpu.VMEM((NB,), jnp.int32),
                   pltpu.VMEM((WT,), jnp.float32),
                   pltpu.VMEM((WT,), jnp.int32),
                   pltpu.VMEM((WT,), jnp.int32),
                   pltpu.VMEM((WCN, WCH), jnp.int32),
                   pltpu.VMEM((WCN, WCH), jnp.int32),
                   pltpu.SemaphoreType.DMA],
)


def kernel(edge_index, num_nodes):
  row = edge_index[0].astype(jnp.int32)
  col = edge_index[1].astype(jnp.int32)
  row_p = jnp.concatenate(
      [row, jnp.full((E_PAD - E,), N_STATIC, jnp.int32)])
  col_p = jnp.concatenate([col, jnp.zeros((E_PAD - E,), jnp.int32)])

  nn = jnp.asarray(num_nodes, jnp.int32)
  starts = jnp.minimum(jnp.arange(N_STATIC, dtype=jnp.int32), nn - 1)
  starts_p = jnp.concatenate(
      [starts, jnp.zeros((NB - N_STATIC,), jnp.int32)])

  keys = jax.random.split(jax.random.key(42), NUM_STEPS)
  u = jnp.stack([jax.random.uniform(k, (N_STATIC,)) for k in keys])
  u_p = jnp.concatenate(
      [u, jnp.zeros((NUM_STEPS, NB - N_STATIC), jnp.float32)],
      axis=1).reshape(-1)

  hist, tots = _hist(row_p)
  rowptr, bases = _base(hist, tots)
  col_sorted = _perm(row_p, col_p, bases)
  paths = _walk(rowptr, col_sorted, u_p, starts_p)

  return tuple(
      jnp.stack([starts, paths[k * NB:k * NB + N_STATIC]], axis=0)
      for k in range(len(KEEP_COLS)))
